# baseline (speedup 1.0000x reference)
import functools

import jax
import jax.numpy as jnp
from jax import lax
from jax.experimental import pallas as pl
from jax.experimental.pallas import tpu as pltpu
from jax.experimental.pallas import tpu_sc as plsc

_NUM_CORES = 2
_NUM_SUBCORES = 16
_NUM_WORKERS = _NUM_CORES * _NUM_SUBCORES
_LANES = 16


@functools.lru_cache(maxsize=None)
def _build(batch: int, side: int, rows_per_group: int):
    length = side * side
    assert batch % _NUM_WORKERS == 0
    rows_per_worker = batch // _NUM_WORKERS
    assert rows_per_worker % rows_per_group == 0
    n_groups = rows_per_worker // rows_per_group
    blocks_per_row = side // _LANES
    R = rows_per_group

    mesh = plsc.VectorSubcoreMesh(
        core_axis_name="c", subcore_axis_name="s")

    @functools.partial(
        pl.kernel,
        out_type=jax.ShapeDtypeStruct((batch, side, side), jnp.float32),
        mesh=mesh,
        compiler_params=pltpu.CompilerParams(
            needs_layout_passes=False, use_tc_tiling_on_sc=False),
        scratch_types=[
            pltpu.VMEM((side, side), jnp.int32),
            pltpu.VMEM((R, length), jnp.float32),
            pltpu.VMEM((R, length), jnp.float32),
            pltpu.VMEM((R, side, side), jnp.float32),
            pltpu.VMEM((R, side, side), jnp.float32),
            pltpu.SemaphoreType.DMA,
            pltpu.SemaphoreType.DMA,
            pltpu.SemaphoreType.DMA,
            pltpu.SemaphoreType.DMA,
        ],
    )
    def hilbert_select(x_hbm, hm_hbm, out_hbm, idx_v,
                       in0, in1, out0, out1, isem0, isem1, osem0, osem1):
        wid = lax.axis_index("s") * _NUM_CORES + lax.axis_index("c")
        row_base = wid * rows_per_worker
        pltpu.sync_copy(hm_hbm, idx_v)

        ins = (in0, in1)
        outs = (out0, out1)
        isems = (isem0, isem1)
        osems = (osem0, osem1)

        in_descs = [None, None]
        out_descs = [None, None]

        in_descs[0] = pltpu.async_copy(
            x_hbm.at[pl.ds(row_base, R)], ins[0], isems[0])

        for g in range(n_groups):
            p = g % 2
            in_descs[p].wait()
            if g + 1 < n_groups:
                in_descs[1 - p] = pltpu.async_copy(
                    x_hbm.at[pl.ds(row_base + (g + 1) * R, R)],
                    ins[1 - p], isems[1 - p])
            if out_descs[p] is not None:
                out_descs[p].wait()

            src = ins[p]
            dst = outs[p]

            @plsc.parallel_loop(0, side, unroll=1)
            def mat_row_body(i):
                for j in range(blocks_per_row):
                    iv = idx_v[i, pl.ds(j * _LANES, _LANES)]
                    for r in range(R):
                        rv = jnp.full((_LANES,), r, jnp.int32)
                        dst[r, i, pl.ds(j * _LANES, _LANES)] = (
                            plsc.load_gather(src, [rv, iv]))

            out_descs[p] = pltpu.async_copy(
                dst, out_hbm.at[pl.ds(row_base + g * R, R)], osems[p])

        out_descs[0].wait()
        out_descs[1].wait()

    return hilbert_select


def kernel(x, hilbert_matrix):
    batch, length = x.shape
    side = hilbert_matrix.shape[0]
    assert side * side == length
    return _build(batch, side, 4)(x, hilbert_matrix)


# speedup vs baseline: 1.0137x; 1.0137x over previous
import functools

import jax
import jax.numpy as jnp
from jax import lax
from jax.experimental import pallas as pl
from jax.experimental.pallas import tpu as pltpu
from jax.experimental.pallas import tpu_sc as plsc

_NUM_CORES = 2
_NUM_SUBCORES = 16
_NUM_WORKERS = _NUM_CORES * _NUM_SUBCORES
_LANES = 16

_TR = 32768
_NSLOT = 3
_SG = 8
_IC = 8


@functools.lru_cache(maxsize=None)
def _build(batch: int, side: int):
    length = side * side
    assert batch == 4096 and side == 64 and length == 4096
    n_tile_rows = batch // 8
    rows_per_worker = batch // _NUM_WORKERS
    trs_per_worker = rows_per_worker // 8

    mesh = plsc.VectorSubcoreMesh(
        core_axis_name="c", subcore_axis_name="s")

    @functools.partial(
        pl.kernel,
        out_type=jax.ShapeDtypeStruct(
            (side, 8, batch // 128, 8, 128), jnp.float32),
        mesh=mesh,
        compiler_params=pltpu.CompilerParams(
            needs_layout_passes=False, use_tc_tiling_on_sc=False),
        scratch_types=[
            pltpu.VMEM((side, side), jnp.int32),
            pltpu.VMEM((length,), jnp.int32),
            pltpu.VMEM((_NSLOT * _TR,), jnp.float32),
            pltpu.VMEM((2, 8, 8, 8, _LANES), jnp.float32),
            pltpu.SemaphoreType.DMA,
            pltpu.SemaphoreType.DMA,
            pltpu.SemaphoreType.DMA,
            pltpu.SemaphoreType.DMA,
            pltpu.SemaphoreType.DMA,
        ],
    )
    def hilbert_select(x_hbm, hm_hbm, out_hbm, hm_v, idx_t, in_ring, out_v,
                       is0, is1, is2, os0, os1):
        wid = lax.axis_index("s") * _NUM_CORES + lax.axis_index("c")
        tr0 = wid * trs_per_worker
        isems = (is0, is1, is2)
        osems = (os0, os1)

        pltpu.sync_copy(hm_hbm, hm_v)

        def transform(c, carry):
            i = c >> 2
            j0 = (c & 3) * _LANES
            d = hm_v[i, pl.ds(j0, _LANES)]
            idx_t[pl.ds(c * _LANES, _LANES)] = ((d >> 7) << 10) + (d & 127)
            return carry

        lax.fori_loop(0, length // _LANES, transform, 0)

        iota16 = lax.iota(jnp.int32, _LANES)
        offb = (iota16 & 7) * 128

        in_descs = {}

        def start_in(k):
            s = k % _NSLOT
            in_descs[s] = pltpu.async_copy(
                x_hbm.at[tr0 + k], in_ring.at[pl.ds(s * _TR, _TR)], isems[s])

        start_in(0)
        start_in(1)
        start_in(2)

        for sg in range(_SG):
            s0 = (2 * sg) % _NSLOT
            s1 = (2 * sg + 1) % _NSLOT
            in_descs[s0].wait()
            in_descs[s1].wait()
            lane_base = jnp.where(
                iota16 < 8, jnp.int32(s0 * _TR), jnp.int32(s1 * _TR)) + offb
            bm0 = sg * _LANES

            def compute_chunk(i0, pp, lane_base=lane_base):
                @plsc.parallel_loop(0, 32, unroll=1)
                def unit(u):
                    i_loc = u >> 2
                    jt2 = u & 3
                    coff = (i0 + i_loc) * side + jt2 * _LANES
                    tv = idx_t[pl.ds(coff, _LANES)]
                    for k in range(_LANES):
                        jt = jt2 * 2 + (k >> 3)
                        j8 = k & 7
                        bc = lax.gather(
                            tv, jnp.full((_LANES, 1), k, jnp.int32),
                            lax.GatherDimensionNumbers(
                                offset_dims=(),
                                collapsed_slice_dims=(0,),
                                start_index_map=(0,)),
                            slice_sizes=(1,),
                            mode=lax.GatherScatterMode.PROMISE_IN_BOUNDS)
                        out_v[pp, i_loc, jt, j8, :] = plsc.load_gather(
                            in_ring, [lane_base + bc])

            def issue_out(i0, pp, bm0=bm0):
                return pltpu.async_copy(
                    out_v.at[pp],
                    out_hbm.at[pl.ds(i0, 8), :, wid, :, pl.ds(bm0, _LANES)],
                    osems[pp])

            def wait_out(pp, bm0=bm0):
                pltpu.make_async_copy(
                    out_v.at[pp],
                    out_hbm.at[pl.ds(0, 8), :, wid, :, pl.ds(bm0, _LANES)],
                    osems[pp]).wait()

            compute_chunk(0, 0)
            issue_out(0, 0)
            compute_chunk(8, 1)
            issue_out(8, 1)

            def ic_body(ic2, carry):
                i0a = ic2 * 16
                wait_out(0)
                compute_chunk(i0a, 0)
                issue_out(i0a, 0)
                wait_out(1)
                compute_chunk(i0a + 8, 1)
                issue_out(i0a + 8, 1)
                return carry

            lax.fori_loop(1, _IC // 2, ic_body, 0)
            wait_out(0)
            wait_out(1)
            if 2 * sg + 3 < trs_per_worker:
                start_in(2 * sg + 3)
            if 2 * sg + 4 < trs_per_worker:
                start_in(2 * sg + 4)

    return hilbert_select


def kernel(x, hilbert_matrix):
    batch, length = x.shape
    side = hilbert_matrix.shape[0]
    assert side * side == length
    x2p = (x.reshape(batch // 8, 8, length // 128, 128)
            .transpose(0, 2, 1, 3)
            .reshape(batch // 8, 8 * length))
    out5 = _build(batch, side)(x2p, hilbert_matrix)
    return out5.transpose(2, 4, 0, 1, 3).reshape(batch, side, side)


# speedup vs baseline: 2.4550x; 2.4217x over previous
import functools

import jax
import jax.numpy as jnp
from jax import lax
from jax.experimental import pallas as pl
from jax.experimental.pallas import tpu as pltpu
from jax.experimental.pallas import tpu_sc as plsc

_NUM_CORES = 2
_NUM_SUBCORES = 16
_NUM_WORKERS = _NUM_CORES * _NUM_SUBCORES
_LANES = 16

_TR = 32768
_NSLOT = 3
_SG = 8
_IC = 8


@functools.lru_cache(maxsize=None)
def _build(batch: int, side: int):
    length = side * side
    assert batch == 4096 and side == 64 and length == 4096
    n_tile_rows = batch // 8
    rows_per_worker = batch // _NUM_WORKERS
    trs_per_worker = rows_per_worker // 8

    mesh = plsc.VectorSubcoreMesh(
        core_axis_name="c", subcore_axis_name="s")

    @functools.partial(
        pl.kernel,
        out_type=jax.ShapeDtypeStruct(
            (side, 8, batch // 128, 8, 128), jnp.float32),
        mesh=mesh,
        compiler_params=pltpu.CompilerParams(
            needs_layout_passes=False, use_tc_tiling_on_sc=False),
        scratch_types=[
            pltpu.VMEM((side, side), jnp.int32),
            pltpu.VMEM((length,), jnp.int32),
            pltpu.VMEM((_NSLOT * _TR,), jnp.float32),
            pltpu.VMEM((2, 8, 8, 8, _LANES), jnp.float32),
            pltpu.SemaphoreType.DMA,
            pltpu.SemaphoreType.DMA,
            pltpu.SemaphoreType.DMA,
            pltpu.SemaphoreType.DMA,
            pltpu.SemaphoreType.DMA,
        ],
    )
    def hilbert_select(x_hbm, hm_hbm, out_hbm, hm_v, idx_t, in_ring, out_v,
                       is0, is1, is2, os0, os1):
        wid = lax.axis_index("s") * _NUM_CORES + lax.axis_index("c")
        tr0 = wid * trs_per_worker
        isems = (is0, is1, is2)
        osems = (os0, os1)

        in_descs = {}

        def start_in(k):
            s = k % _NSLOT
            in_descs[s] = pltpu.async_copy(
                x_hbm.at[tr0 + k], in_ring.at[pl.ds(s * _TR, _TR)], isems[s])

        start_in(0)
        start_in(1)
        start_in(2)

        pltpu.sync_copy(hm_hbm, hm_v)

        def transform(c, carry):
            i = c >> 2
            j0 = (c & 3) * _LANES
            d = hm_v[i, pl.ds(j0, _LANES)]
            idx_t[pl.ds(c * _LANES, _LANES)] = ((d >> 7) << 10) + (d & 127)
            return carry

        lax.fori_loop(0, length // _LANES, transform, 0)

        iota16 = lax.iota(jnp.int32, _LANES)
        out_v5 = out_v
        jt_hi = iota16 >> 3
        j8v = iota16 & 7

        out_pending = [False, False]

        def wait_out_final(pp):
            pltpu.make_async_copy(
                out_v5.at[pp],
                out_hbm.at[pl.ds(0, 8), :, wid, :, pl.ds(0, _LANES)],
                osems[pp]).wait()

        for sg in range(_SG):
            s0 = (2 * sg) % _NSLOT
            s1 = (2 * sg + 1) % _NSLOT
            in_descs[s0].wait()
            in_descs[s1].wait()
            rowoff = [
                (s0 if b < 8 else s1) * _TR + (b % 8) * 128
                for b in range(_LANES)
            ]
            bm0 = sg * _LANES

            def compute_chunk(i0, pp):
                @plsc.parallel_loop(0, 32, unroll=1)
                def unit(u):
                    i_loc = u >> 2
                    jt2 = u & 3
                    coff = (i0 + i_loc) * side + jt2 * _LANES
                    tv = idx_t[pl.ds(coff, _LANES)]
                    ppv = jnp.full((_LANES,), pp, jnp.int32)
                    ilv = jnp.full((_LANES,), i_loc, jnp.int32)
                    jtv = jt_hi + 2 * jt2
                    for b in range(_LANES):
                        vals = plsc.load_gather(
                            in_ring, [tv + jnp.int32(rowoff[b])])
                        plsc.store_scatter(
                            out_v, [ppv, ilv, jtv, j8v,
                                    jnp.full((_LANES,), b, jnp.int32)], vals)

            def issue_out(i0, pp, bm0=bm0):
                return pltpu.async_copy(
                    out_v5.at[pp],
                    out_hbm.at[pl.ds(i0, 8), :, wid, :, pl.ds(bm0, _LANES)],
                    osems[pp])

            def wait_out(pp, bm0=bm0):
                pltpu.make_async_copy(
                    out_v5.at[pp],
                    out_hbm.at[pl.ds(0, 8), :, wid, :, pl.ds(bm0, _LANES)],
                    osems[pp]).wait()

            if out_pending[0]:
                wait_out(0)
            compute_chunk(0, 0)
            issue_out(0, 0)
            if out_pending[1]:
                wait_out(1)
            compute_chunk(8, 1)
            issue_out(8, 1)

            def ic_body(ic2, carry):
                i0a = ic2 * 16
                wait_out(0)
                compute_chunk(i0a, 0)
                issue_out(i0a, 0)
                wait_out(1)
                compute_chunk(i0a + 8, 1)
                issue_out(i0a + 8, 1)
                return carry

            lax.fori_loop(1, _IC // 2, ic_body, 0)
            out_pending[0] = out_pending[1] = True
            if 2 * sg + 3 < trs_per_worker:
                start_in(2 * sg + 3)
            if 2 * sg + 4 < trs_per_worker:
                start_in(2 * sg + 4)

        wait_out_final(0)
        wait_out_final(1)

    return hilbert_select


def kernel(x, hilbert_matrix):
    batch, length = x.shape
    side = hilbert_matrix.shape[0]
    assert side * side == length
    x2p = (x.reshape(batch // 8, 8, length // 128, 128)
            .transpose(0, 2, 1, 3)
            .reshape(batch // 8, 8 * length))
    out5 = _build(batch, side)(x2p, hilbert_matrix)
    return out5.transpose(2, 4, 0, 1, 3).reshape(batch, side, side)
